# baseline (speedup 1.0000x reference)
import functools

import jax
import jax.numpy as jnp
from jax import lax
from jax.experimental import pallas as pl
from jax.experimental.pallas import tpu as pltpu
from jax.experimental.pallas import tpu_sc as plsc

N_NODES = 10000
D = 128
N_EDGES = 320000

NC = 2
NS = 16
NW = NC * NS

E_PER_TILE = N_EDGES // NW
CHUNK = 32
N_CHUNKS = 312
TAIL = E_PER_TILE - N_CHUNKS * CHUNK

N_PAD = 10240
ROWS_PER_TILE = N_PAD // NS

NBUF = 8


def _sc_aggregate(x, src, dst):
  mesh = plsc.VectorSubcoreMesh(
      core_axis_name="c", subcore_axis_name="s", num_cores=NC,
      num_subcores=NS)

  @functools.partial(
      pl.kernel,
      out_type=jax.ShapeDtypeStruct((NC, N_PAD, D), jnp.float32),
      mesh=mesh,
      scratch_types=[
          pltpu.VMEM((E_PER_TILE,), jnp.int32),
          [pltpu.VMEM((CHUNK,), jnp.int32) for _ in range(NBUF)],
          [pltpu.VMEM((CHUNK, D), jnp.float32) for _ in range(NBUF)],
          pltpu.VMEM((TAIL,), jnp.int32),
          pltpu.VMEM((TAIL,), jnp.int32),
          pltpu.VMEM((TAIL, D), jnp.float32),
          pltpu.VMEM_SHARED((N_PAD, D), jnp.float32),
          pltpu.SemaphoreType.DMA,
          pltpu.SemaphoreType.DMA,
          [pltpu.SemaphoreType.DMA for _ in range(NBUF)],
          [pltpu.SemaphoreType.DMA for _ in range(NBUF)],
          [pltpu.SemaphoreType.DMA for _ in range(NBUF)],
      ],
  )
  def agg_kernel(x_hbm, src_hbm, dst_hbm, out_hbm,
                 src_all, dst_slot, rows, tsrc, tdst, trows, acc,
                 sem_i, sem_z, sem_d, sem_g, sem_s):
    c = lax.axis_index("c")
    s = lax.axis_index("s")
    wid = s * NC + c
    base = wid * E_PER_TILE
    tail0 = base + N_CHUNKS * CHUNK

    pltpu.async_copy(src_hbm.at[pl.ds(base, E_PER_TILE)], src_all, sem_i)
    pltpu.async_copy(src_hbm.at[pl.ds(tail0, TAIL)], tsrc, sem_i)
    pltpu.async_copy(dst_hbm.at[pl.ds(tail0, TAIL)], tdst, sem_i)

    zeros16 = jnp.zeros((16,), jnp.float32)

    def zrow(i, _):
      for j in range(D // 16):
        rows[7][i, pl.ds(j * 16, 16)] = zeros16
      return 0

    lax.fori_loop(0, CHUNK, zrow, 0)
    r0 = s * ROWS_PER_TILE
    for k in range(ROWS_PER_TILE // CHUNK):
      pltpu.async_copy(rows[7], acc.at[pl.ds(r0 + k * CHUNK, CHUNK), :],
                       sem_z)

    def load_dst(i, sl):
      pltpu.async_copy(dst_hbm.at[pl.ds(base + i * CHUNK, CHUNK)],
                       dst_slot[sl], sem_d[sl])

    def wait_dst(sl):
      pltpu.make_async_copy(dst_hbm.at[pl.ds(base, CHUNK)],
                            dst_slot[sl], sem_d[sl]).wait()

    def start_gather(i, rb):
      pltpu.async_copy(
          x_hbm.at[src_all.at[pl.ds(i * CHUNK, CHUNK)]], rows[rb],
          sem_g[rb])

    def wait_gather(rb):
      pltpu.make_async_copy(
          x_hbm.at[src_all.at[pl.ds(0, CHUNK)]], rows[rb],
          sem_g[rb]).wait()

    def start_scatter(rb, sl):
      pltpu.async_copy(rows[rb], acc.at[dst_slot[sl]], sem_s[rb],
                       add=True)

    def wait_scatter(rb):
      pltpu.make_async_copy(rows[rb], acc.at[dst_slot[0]],
                            sem_s[rb]).wait()

    pltpu.make_async_copy(src_hbm.at[pl.ds(base, E_PER_TILE)], src_all,
                          sem_i).wait()
    for sl in range(4):
      load_dst(sl, sl)
    for bb in range(7):
      start_gather(bb, bb)

    for k in range(ROWS_PER_TILE // CHUNK):
      pltpu.make_async_copy(rows[7],
                            acc.at[pl.ds(r0, CHUNK), :], sem_z).wait()
    plsc.subcore_barrier()

    NG = N_CHUNKS // NBUF
    LAST = NG - 1

    def oct_body(g, _):
      for b in range(NBUF):
        i = g * NBUF + b
        bn = (b + 7) % NBUF
        wait_gather(b)
        wait_dst(b)
        start_scatter(b, b)
        if b >= 1:
          wait_scatter(bn)
        else:
          @pl.when(g > 0)
          def _():
            wait_scatter(bn)
        if b == 0:
          start_gather(i + 7, bn)
        else:
          @pl.when(g < LAST)
          def _():
            start_gather(i + 7, bn)
        if b <= 3:
          load_dst(i + 4, (b + 4) % NBUF)
        else:
          @pl.when(g < LAST)
          def _():
            load_dst(i + 4, (b + 4) % NBUF)
      return 0

    lax.fori_loop(0, NG, oct_body, 0)

    pltpu.make_async_copy(src_hbm.at[pl.ds(tail0, TAIL)], tsrc,
                          sem_i).wait()
    pltpu.make_async_copy(dst_hbm.at[pl.ds(tail0, TAIL)], tdst,
                          sem_i).wait()
    pltpu.sync_copy(x_hbm.at[tsrc], trows)
    pltpu.sync_copy(trows, acc.at[tdst], add=True)

    wait_scatter(7)
    plsc.subcore_barrier()

    pltpu.sync_copy(acc.at[pl.ds(r0, ROWS_PER_TILE), :],
                    out_hbm.at[c, pl.ds(r0, ROWS_PER_TILE), :])

  return agg_kernel(x, src, dst)


BLK = 2000


def _mlp_block(x_ref, p0_ref, p1_ref, w1_ref, b1_ref, w2_ref, b2_ref,
               out_ref):
  h = x_ref[...] + p0_ref[0] + p1_ref[0]
  h = jnp.dot(h, w1_ref[...], preferred_element_type=jnp.float32)
  h = jnp.maximum(h + b1_ref[...], 0.0)
  out_ref[...] = (
      jnp.dot(h, w2_ref[...], preferred_element_type=jnp.float32)
      + b2_ref[...])


def _mlp(x, partials, W1, b1, W2, b2):
  grid = (N_NODES // BLK,)
  row_spec = pl.BlockSpec((BLK, D), lambda i: (i, 0))
  p0_spec = pl.BlockSpec((1, BLK, D), lambda i: (0, i, 0))
  p1_spec = pl.BlockSpec((1, BLK, D), lambda i: (1, i, 0))
  full = pl.BlockSpec((D, D), lambda i: (0, 0))
  vec = pl.BlockSpec((1, D), lambda i: (0, 0))
  return pl.pallas_call(
      _mlp_block,
      grid=grid,
      in_specs=[row_spec, p0_spec, p1_spec, full, vec, full, vec],
      out_specs=row_spec,
      out_shape=jax.ShapeDtypeStruct((N_NODES, D), jnp.float32),
  )(x, partials, partials, W1, b1.reshape(1, D), W2, b2.reshape(1, D))


@jax.jit
def kernel(x, edge_index, W1, b1, W2, b2):
  src = edge_index[0].astype(jnp.int32)
  dst = edge_index[1].astype(jnp.int32)
  partials = _sc_aggregate(x, src, dst)
  return _mlp(x, partials, W1, b1, W2, b2)


# speedup vs baseline: 1.0031x; 1.0031x over previous
import functools

import jax
import jax.numpy as jnp
from jax import lax
from jax.experimental import pallas as pl
from jax.experimental.pallas import tpu as pltpu
from jax.experimental.pallas import tpu_sc as plsc

N_NODES = 10000
D = 128
N_EDGES = 320000

NC = 2
NS = 16
NW = NC * NS

E_PER_TILE = N_EDGES // NW
CHUNK = 32
N_CHUNKS = 312
TAIL = E_PER_TILE - N_CHUNKS * CHUNK

N_PAD = 10240
ROWS_PER_TILE = N_PAD // NS

NBUF = 8


def _sc_aggregate(x, src, dst):
  mesh = plsc.VectorSubcoreMesh(
      core_axis_name="c", subcore_axis_name="s", num_cores=NC,
      num_subcores=NS)

  @functools.partial(
      pl.kernel,
      out_type=jax.ShapeDtypeStruct((NC, N_PAD, D), jnp.float32),
      mesh=mesh,
      scratch_types=[
          pltpu.VMEM((E_PER_TILE,), jnp.int32),
          [pltpu.VMEM((CHUNK,), jnp.int32) for _ in range(NBUF)],
          [pltpu.VMEM((CHUNK, D), jnp.float32) for _ in range(NBUF)],
          pltpu.VMEM((TAIL,), jnp.int32),
          pltpu.VMEM((TAIL,), jnp.int32),
          pltpu.VMEM((TAIL, D), jnp.float32),
          pltpu.VMEM_SHARED((N_PAD, D), jnp.float32),
          pltpu.SemaphoreType.DMA,
          pltpu.SemaphoreType.DMA,
          [pltpu.SemaphoreType.DMA for _ in range(NBUF)],
          [pltpu.SemaphoreType.DMA for _ in range(NBUF)],
          [pltpu.SemaphoreType.DMA for _ in range(NBUF)],
      ],
  )
  def agg_kernel(x_hbm, src_hbm, dst_hbm, out_hbm,
                 src_all, dst_slot, rows, tsrc, tdst, trows, acc,
                 sem_i, sem_z, sem_d, sem_g, sem_s):
    c = lax.axis_index("c")
    s = lax.axis_index("s")
    wid = s * NC + c
    base = wid * E_PER_TILE
    tail0 = base + N_CHUNKS * CHUNK

    pltpu.async_copy(src_hbm.at[pl.ds(base, E_PER_TILE)], src_all, sem_i)
    pltpu.async_copy(src_hbm.at[pl.ds(tail0, TAIL)], tsrc, sem_i)
    pltpu.async_copy(dst_hbm.at[pl.ds(tail0, TAIL)], tdst, sem_i)

    zeros16 = jnp.zeros((16,), jnp.float32)

    def zrow(i, _):
      for j in range(D // 16):
        rows[7][i, pl.ds(j * 16, 16)] = zeros16
      return 0

    lax.fori_loop(0, CHUNK, zrow, 0)
    r0 = s * ROWS_PER_TILE
    for k in range(ROWS_PER_TILE // CHUNK):
      pltpu.async_copy(rows[7], acc.at[pl.ds(r0 + k * CHUNK, CHUNK), :],
                       sem_z)

    def load_dst(i, sl):
      pltpu.async_copy(dst_hbm.at[pl.ds(base + i * CHUNK, CHUNK)],
                       dst_slot[sl], sem_d[sl])

    def wait_dst(sl):
      pltpu.make_async_copy(dst_hbm.at[pl.ds(base, CHUNK)],
                            dst_slot[sl], sem_d[sl]).wait()

    def start_gather(i, rb):
      pltpu.async_copy(
          x_hbm.at[src_all.at[pl.ds(i * CHUNK, CHUNK)]], rows[rb],
          sem_g[rb])

    def wait_gather(rb):
      pltpu.make_async_copy(
          x_hbm.at[src_all.at[pl.ds(0, CHUNK)]], rows[rb],
          sem_g[rb]).wait()

    def start_scatter(rb, sl):
      pltpu.async_copy(rows[rb], acc.at[dst_slot[sl]], sem_s[rb],
                       add=True)

    def wait_scatter(rb):
      pltpu.make_async_copy(rows[rb], acc.at[dst_slot[0]],
                            sem_s[rb]).wait()

    pltpu.make_async_copy(src_hbm.at[pl.ds(base, E_PER_TILE)], src_all,
                          sem_i).wait()
    for sl in range(4):
      load_dst(sl, sl)
    for bb in range(6):
      start_gather(bb, bb)

    for k in range(ROWS_PER_TILE // CHUNK):
      pltpu.make_async_copy(rows[7],
                            acc.at[pl.ds(r0, CHUNK), :], sem_z).wait()
    plsc.subcore_barrier()

    NG = N_CHUNKS // NBUF
    LAST = NG - 1

    def oct_body(g, _):
      for b in range(NBUF):
        i = g * NBUF + b
        bn = (b + 6) % NBUF
        wait_gather(b)
        wait_dst(b)
        start_scatter(b, b)
        if b >= 2:
          wait_scatter(bn)
        else:
          @pl.when(g > 0)
          def _():
            wait_scatter(bn)
        if b <= 1:
          start_gather(i + 6, bn)
        else:
          @pl.when(g < LAST)
          def _():
            start_gather(i + 6, bn)
        if b <= 3:
          load_dst(i + 4, (b + 4) % NBUF)
        else:
          @pl.when(g < LAST)
          def _():
            load_dst(i + 4, (b + 4) % NBUF)
      return 0

    lax.fori_loop(0, NG, oct_body, 0)

    pltpu.make_async_copy(src_hbm.at[pl.ds(tail0, TAIL)], tsrc,
                          sem_i).wait()
    pltpu.make_async_copy(dst_hbm.at[pl.ds(tail0, TAIL)], tdst,
                          sem_i).wait()
    pltpu.sync_copy(x_hbm.at[tsrc], trows)
    pltpu.sync_copy(trows, acc.at[tdst], add=True)

    wait_scatter(6)
    wait_scatter(7)
    plsc.subcore_barrier()

    pltpu.sync_copy(acc.at[pl.ds(r0, ROWS_PER_TILE), :],
                    out_hbm.at[c, pl.ds(r0, ROWS_PER_TILE), :])

  return agg_kernel(x, src, dst)


BLK = 2000


def _mlp_block(x_ref, p0_ref, p1_ref, w1_ref, b1_ref, w2_ref, b2_ref,
               out_ref):
  h = x_ref[...] + p0_ref[0] + p1_ref[0]
  h = jnp.dot(h, w1_ref[...], preferred_element_type=jnp.float32)
  h = jnp.maximum(h + b1_ref[...], 0.0)
  out_ref[...] = (
      jnp.dot(h, w2_ref[...], preferred_element_type=jnp.float32)
      + b2_ref[...])


def _mlp(x, partials, W1, b1, W2, b2):
  grid = (N_NODES // BLK,)
  row_spec = pl.BlockSpec((BLK, D), lambda i: (i, 0))
  p0_spec = pl.BlockSpec((1, BLK, D), lambda i: (0, i, 0))
  p1_spec = pl.BlockSpec((1, BLK, D), lambda i: (1, i, 0))
  full = pl.BlockSpec((D, D), lambda i: (0, 0))
  vec = pl.BlockSpec((1, D), lambda i: (0, 0))
  return pl.pallas_call(
      _mlp_block,
      grid=grid,
      in_specs=[row_spec, p0_spec, p1_spec, full, vec, full, vec],
      out_specs=row_spec,
      out_shape=jax.ShapeDtypeStruct((N_NODES, D), jnp.float32),
  )(x, partials, partials, W1, b1.reshape(1, D), W2, b2.reshape(1, D))


@jax.jit
def kernel(x, edge_index, W1, b1, W2, b2):
  src = edge_index[0].astype(jnp.int32)
  dst = edge_index[1].astype(jnp.int32)
  partials = _sc_aggregate(x, src, dst)
  return _mlp(x, partials, W1, b1, W2, b2)


# speedup vs baseline: 1.0912x; 1.0878x over previous
import functools

import jax
import jax.numpy as jnp
from jax import lax
from jax.experimental import pallas as pl
from jax.experimental.pallas import tpu as pltpu
from jax.experimental.pallas import tpu_sc as plsc

N_NODES = 10000
D = 128
N_EDGES = 320000

NC = 2
NS = 16
NW = NC * NS

E_PER_TILE = N_EDGES // NW
CHUNK = 32
N_CHUNKS = 312
TAIL = E_PER_TILE - N_CHUNKS * CHUNK

N_PAD = 10240
ROWS_PER_TILE = N_PAD // NS

NBUF = 8


def _sc_aggregate(x, edge_flat):
  mesh = plsc.VectorSubcoreMesh(
      core_axis_name="c", subcore_axis_name="s", num_cores=NC,
      num_subcores=NS)

  @functools.partial(
      pl.kernel,
      out_type=jax.ShapeDtypeStruct((NC, N_PAD, D), jnp.float32),
      mesh=mesh,
      scratch_types=[
          pltpu.VMEM((E_PER_TILE,), jnp.int32),
          [pltpu.VMEM((CHUNK,), jnp.int32) for _ in range(NBUF)],
          [pltpu.VMEM((CHUNK, D), jnp.float32) for _ in range(NBUF)],
          pltpu.VMEM((TAIL,), jnp.int32),
          pltpu.VMEM((TAIL,), jnp.int32),
          pltpu.VMEM((TAIL, D), jnp.float32),
          pltpu.VMEM_SHARED((N_PAD, D), jnp.float32),
          pltpu.SemaphoreType.DMA,
          pltpu.SemaphoreType.DMA,
          [pltpu.SemaphoreType.DMA for _ in range(NBUF)],
          [pltpu.SemaphoreType.DMA for _ in range(NBUF)],
          [pltpu.SemaphoreType.DMA for _ in range(NBUF)],
      ],
  )
  def agg_kernel(x_hbm, e_hbm, out_hbm,
                 src_all, dst_slot, rows, tsrc, tdst, trows, acc,
                 sem_i, sem_z, sem_d, sem_g, sem_s):
    c = lax.axis_index("c")
    s = lax.axis_index("s")
    wid = s * NC + c
    base = wid * E_PER_TILE
    tail0 = base + N_CHUNKS * CHUNK

    pltpu.async_copy(e_hbm.at[pl.ds(base, E_PER_TILE)], src_all, sem_i)
    pltpu.async_copy(e_hbm.at[pl.ds(tail0, TAIL)], tsrc, sem_i)
    pltpu.async_copy(e_hbm.at[pl.ds(N_EDGES + tail0, TAIL)], tdst, sem_i)

    zeros16 = jnp.zeros((16,), jnp.float32)

    def zrow(i, _):
      for j in range(D // 16):
        rows[7][i, pl.ds(j * 16, 16)] = zeros16
      return 0

    lax.fori_loop(0, CHUNK, zrow, 0)
    r0 = s * ROWS_PER_TILE
    for k in range(ROWS_PER_TILE // CHUNK):
      pltpu.async_copy(rows[7], acc.at[pl.ds(r0 + k * CHUNK, CHUNK), :],
                       sem_z)

    def load_dst(i, sl):
      pltpu.async_copy(e_hbm.at[pl.ds(N_EDGES + base + i * CHUNK, CHUNK)],
                       dst_slot[sl], sem_d[sl])

    def wait_dst(sl):
      pltpu.make_async_copy(e_hbm.at[pl.ds(base, CHUNK)],
                            dst_slot[sl], sem_d[sl]).wait()

    def start_gather(i, rb):
      pltpu.async_copy(
          x_hbm.at[src_all.at[pl.ds(i * CHUNK, CHUNK)]], rows[rb],
          sem_g[rb])

    def wait_gather(rb):
      pltpu.make_async_copy(
          x_hbm.at[src_all.at[pl.ds(0, CHUNK)]], rows[rb],
          sem_g[rb]).wait()

    def start_scatter(rb, sl):
      pltpu.async_copy(rows[rb], acc.at[dst_slot[sl]], sem_s[rb],
                       add=True)

    def wait_scatter(rb):
      pltpu.make_async_copy(rows[rb], acc.at[dst_slot[0]],
                            sem_s[rb]).wait()

    pltpu.make_async_copy(e_hbm.at[pl.ds(base, E_PER_TILE)], src_all,
                          sem_i).wait()
    for sl in range(4):
      load_dst(sl, sl)
    for bb in range(6):
      start_gather(bb, bb)

    for k in range(ROWS_PER_TILE // CHUNK):
      pltpu.make_async_copy(rows[7],
                            acc.at[pl.ds(r0, CHUNK), :], sem_z).wait()
    plsc.subcore_barrier()

    NG = N_CHUNKS // NBUF
    LAST = NG - 1

    def oct_body(g, _):
      for b in range(NBUF):
        i = g * NBUF + b
        bn = (b + 6) % NBUF
        wait_gather(b)
        wait_dst(b)
        start_scatter(b, b)
        if b >= 2:
          wait_scatter(bn)
        else:
          @pl.when(g > 0)
          def _():
            wait_scatter(bn)
        if b <= 1:
          start_gather(i + 6, bn)
        else:
          @pl.when(g < LAST)
          def _():
            start_gather(i + 6, bn)
        if b <= 3:
          load_dst(i + 4, (b + 4) % NBUF)
        else:
          @pl.when(g < LAST)
          def _():
            load_dst(i + 4, (b + 4) % NBUF)
      return 0

    lax.fori_loop(0, NG, oct_body, 0)

    pltpu.make_async_copy(e_hbm.at[pl.ds(tail0, TAIL)], tsrc,
                          sem_i).wait()
    pltpu.make_async_copy(e_hbm.at[pl.ds(N_EDGES + tail0, TAIL)], tdst,
                          sem_i).wait()
    pltpu.sync_copy(x_hbm.at[tsrc], trows)
    pltpu.sync_copy(trows, acc.at[tdst], add=True)

    wait_scatter(6)
    wait_scatter(7)
    plsc.subcore_barrier()

    pltpu.sync_copy(acc.at[pl.ds(r0, ROWS_PER_TILE), :],
                    out_hbm.at[c, pl.ds(r0, ROWS_PER_TILE), :])

  return agg_kernel(x, edge_flat)


BLK = 2000


def _mlp_block(x_ref, p0_ref, p1_ref, w1_ref, b1_ref, w2_ref, b2_ref,
               out_ref):
  h = x_ref[...] + p0_ref[0] + p1_ref[0]
  h = jnp.dot(h, w1_ref[...], preferred_element_type=jnp.float32)
  h = jnp.maximum(h + b1_ref[...], 0.0)
  out_ref[...] = (
      jnp.dot(h, w2_ref[...], preferred_element_type=jnp.float32)
      + b2_ref[...])


def _mlp(x, partials, W1, b1, W2, b2):
  grid = (N_NODES // BLK,)
  row_spec = pl.BlockSpec((BLK, D), lambda i: (i, 0))
  p0_spec = pl.BlockSpec((1, BLK, D), lambda i: (0, i, 0))
  p1_spec = pl.BlockSpec((1, BLK, D), lambda i: (1, i, 0))
  full = pl.BlockSpec((D, D), lambda i: (0, 0))
  vec = pl.BlockSpec((1, D), lambda i: (0, 0))
  return pl.pallas_call(
      _mlp_block,
      grid=grid,
      in_specs=[row_spec, p0_spec, p1_spec, full, vec, full, vec],
      out_specs=row_spec,
      out_shape=jax.ShapeDtypeStruct((N_NODES, D), jnp.float32),
  )(x, partials, partials, W1, b1.reshape(1, D), W2, b2.reshape(1, D))


@jax.jit
def kernel(x, edge_index, W1, b1, W2, b2):
  partials = _sc_aggregate(x, edge_index.astype(jnp.int32).ravel())
  return _mlp(x, partials, W1, b1, W2, b2)
